# baseline (speedup 1.0000x reference)
import functools

import jax
import jax.numpy as jnp
from jax import lax
from jax.experimental import pallas as pl
from jax.experimental.pallas import tpu as pltpu, tpu_sc as plsc

S, C, L, T, B = 10, 100, 4, 6, 16384
TABLE_N = S * C * L * T

_info = plsc.get_sparse_core_info()
_NC, _NS, _LANES = _info.num_cores, _info.num_subcores, _info.num_lanes
_NW = _NC * _NS
_BPW = B // _NW
_STEPS = _BPW // _LANES

_mesh = plsc.VectorSubcoreMesh(core_axis_name="c", subcore_axis_name="s")


@functools.partial(
    pl.kernel,
    mesh=_mesh,
    out_type=jax.ShapeDtypeStruct((B,), jnp.float32),
    compiler_params=pltpu.CompilerParams(needs_layout_passes=False),
    scratch_types=[
        pltpu.VMEM((TABLE_N,), jnp.float32),
        pltpu.VMEM((_BPW,), jnp.int32),
        pltpu.VMEM((_BPW,), jnp.int32),
        pltpu.VMEM((_BPW,), jnp.int32),
        pltpu.VMEM((_BPW,), jnp.int32),
        pltpu.VMEM((_BPW,), jnp.float32),
    ],
)
def _balancer_gather(table_hbm, src_hbm, cnt_hbm, lab_hbm, vt_hbm, out_hbm,
                     table_v, src_v, cnt_v, lab_v, vt_v, out_v):
    wid = lax.axis_index("s") * _NC + lax.axis_index("c")
    base = wid * _BPW

    pltpu.sync_copy(table_hbm, table_v)
    pltpu.sync_copy(src_hbm.at[pl.ds(base, _BPW)], src_v)
    pltpu.sync_copy(cnt_hbm.at[pl.ds(base, _BPW)], cnt_v)
    pltpu.sync_copy(lab_hbm.at[pl.ds(base, _BPW)], lab_v)
    pltpu.sync_copy(vt_hbm.at[pl.ds(base, _BPW)], vt_v)

    for i in range(_STEPS):
        sl = pl.ds(i * _LANES, _LANES)
        lin = (src_v[sl] * (C * L * T) + cnt_v[sl] * (L * T)
               + lab_v[sl] * T + vt_v[sl])
        out_v[sl] = plsc.load_gather(table_v, [lin])

    pltpu.sync_copy(out_v, out_hbm.at[pl.ds(base, _BPW)])


def kernel(label_balancing_weights_sclt, sources, alt_counts, labels, variant_types):
    table = label_balancing_weights_sclt.reshape(-1)
    return _balancer_gather(table, sources, alt_counts, labels, variant_types)


# speedup vs baseline: 1.0877x; 1.0877x over previous
import functools

import jax
import jax.numpy as jnp
from jax import lax
from jax.experimental import pallas as pl
from jax.experimental.pallas import tpu as pltpu, tpu_sc as plsc

S, C, L, T, B = 10, 100, 4, 6, 16384
TABLE_N = S * C * L * T

_info = plsc.get_sparse_core_info()
_NC, _NS, _LANES = _info.num_cores, _info.num_subcores, _info.num_lanes
_NW = _NC * _NS
_BPW = B // _NW
_STEPS = _BPW // _LANES

_mesh = plsc.VectorSubcoreMesh(core_axis_name="c", subcore_axis_name="s")


@functools.partial(
    pl.kernel,
    mesh=_mesh,
    out_type=jax.ShapeDtypeStruct((B,), jnp.float32),
    compiler_params=pltpu.CompilerParams(needs_layout_passes=False),
    scratch_types=[
        pltpu.VMEM((TABLE_N,), jnp.float32),
        pltpu.VMEM((_BPW,), jnp.int32),
        pltpu.VMEM((_BPW,), jnp.int32),
        pltpu.VMEM((_BPW,), jnp.int32),
        pltpu.VMEM((_BPW,), jnp.int32),
        pltpu.VMEM((_BPW,), jnp.float32),
        pltpu.SemaphoreType.DMA,
    ],
)
def _balancer_gather(table_hbm, src_hbm, cnt_hbm, lab_hbm, vt_hbm, out_hbm,
                     table_v, src_v, cnt_v, lab_v, vt_v, out_v, sem):
    wid = lax.axis_index("s") * _NC + lax.axis_index("c")
    base = wid * _BPW

    sl_in = pl.ds(base, _BPW)
    copies = [
        pltpu.async_copy(table_hbm, table_v, sem),
        pltpu.async_copy(src_hbm.at[sl_in], src_v, sem),
        pltpu.async_copy(cnt_hbm.at[sl_in], cnt_v, sem),
        pltpu.async_copy(lab_hbm.at[sl_in], lab_v, sem),
        pltpu.async_copy(vt_hbm.at[sl_in], vt_v, sem),
    ]
    for cp in copies:
        cp.wait()

    for i in range(_STEPS):
        sl = pl.ds(i * _LANES, _LANES)
        lin = (src_v[sl] * (C * L * T) + cnt_v[sl] * (L * T)
               + lab_v[sl] * T + vt_v[sl])
        out_v[sl] = plsc.load_gather(table_v, [lin])

    pltpu.sync_copy(out_v, out_hbm.at[pl.ds(base, _BPW)])


def kernel(label_balancing_weights_sclt, sources, alt_counts, labels, variant_types):
    table = label_balancing_weights_sclt.reshape(-1)
    return _balancer_gather(table, sources, alt_counts, labels, variant_types)


# speedup vs baseline: 1.1547x; 1.0616x over previous
import functools

import jax
import jax.numpy as jnp
from jax import lax
from jax.experimental import pallas as pl
from jax.experimental.pallas import tpu as pltpu, tpu_sc as plsc

S, C, L, T, B = 10, 100, 4, 6, 16384
TABLE_N = S * C * L * T

_info = plsc.get_sparse_core_info()
_NC, _NS, _LANES = _info.num_cores, _info.num_subcores, _info.num_lanes
_NW = _NC * _NS
_BPW = B // _NW
_STEPS = _BPW // _LANES

_mesh = plsc.VectorSubcoreMesh(core_axis_name="c", subcore_axis_name="s")


@functools.partial(
    pl.kernel,
    mesh=_mesh,
    out_type=jax.ShapeDtypeStruct((B,), jnp.float32),
    compiler_params=pltpu.CompilerParams(needs_layout_passes=False),
    scratch_types=[
        pltpu.VMEM((_BPW,), jnp.int32),
        pltpu.VMEM((_BPW,), jnp.int32),
        pltpu.VMEM((_BPW,), jnp.int32),
        pltpu.VMEM((_BPW,), jnp.int32),
        pltpu.VMEM((_BPW,), jnp.int32),
        pltpu.VMEM((_BPW,), jnp.float32),
        pltpu.SemaphoreType.DMA,
    ],
)
def _balancer_gather(table_hbm, src_hbm, cnt_hbm, lab_hbm, vt_hbm, out_hbm,
                     src_v, cnt_v, lab_v, vt_v, lin_v, out_v, sem):
    wid = lax.axis_index("s") * _NC + lax.axis_index("c")
    base = wid * _BPW

    sl_in = pl.ds(base, _BPW)
    copies = [
        pltpu.async_copy(src_hbm.at[sl_in], src_v, sem),
        pltpu.async_copy(cnt_hbm.at[sl_in], cnt_v, sem),
        pltpu.async_copy(lab_hbm.at[sl_in], lab_v, sem),
        pltpu.async_copy(vt_hbm.at[sl_in], vt_v, sem),
    ]
    for cp in copies:
        cp.wait()

    for i in range(_STEPS):
        sl = pl.ds(i * _LANES, _LANES)
        lin_v[sl] = (src_v[sl] * (C * L * T) + cnt_v[sl] * (L * T)
                     + lab_v[sl] * T + vt_v[sl])

    pltpu.async_copy(table_hbm.at[lin_v], out_v, sem).wait()
    pltpu.sync_copy(out_v, out_hbm.at[pl.ds(base, _BPW)])


def kernel(label_balancing_weights_sclt, sources, alt_counts, labels, variant_types):
    table = label_balancing_weights_sclt.reshape(-1)
    return _balancer_gather(table, sources, alt_counts, labels, variant_types)


# speedup vs baseline: 1.1578x; 1.0026x over previous
import functools

import jax
import jax.numpy as jnp
from jax import lax
from jax.experimental import pallas as pl
from jax.experimental.pallas import tpu as pltpu, tpu_sc as plsc

S, C, L, T, B = 10, 100, 4, 6, 16384
TABLE_N = S * C * L * T

_info = plsc.get_sparse_core_info()
_NC, _NS, _LANES = _info.num_cores, _info.num_subcores, _info.num_lanes
_NW = _NC * _NS
_BPW = B // _NW
_STEPS = _BPW // _LANES

_mesh = plsc.VectorSubcoreMesh(core_axis_name="c", subcore_axis_name="s")


@functools.partial(
    pl.kernel,
    mesh=_mesh,
    out_type=jax.ShapeDtypeStruct((B,), jnp.float32),
    compiler_params=pltpu.CompilerParams(
        needs_layout_passes=False, use_tc_tiling_on_sc=False),
    scratch_types=[
        pltpu.VMEM((_BPW,), jnp.int32),
        pltpu.VMEM((_BPW,), jnp.int32),
        pltpu.VMEM((_BPW,), jnp.int32),
        pltpu.VMEM((_BPW,), jnp.int32),
        pltpu.VMEM((_BPW,), jnp.int32),
        pltpu.VMEM((_BPW,), jnp.float32),
        pltpu.SemaphoreType.DMA,
    ],
)
def _balancer_gather(table_hbm, src_hbm, cnt_hbm, lab_hbm, vt_hbm, out_hbm,
                     src_v, cnt_v, lab_v, vt_v, lin_v, out_v, sem):
    wid = lax.axis_index("s") * _NC + lax.axis_index("c")
    base = wid * _BPW

    sl_in = pl.ds(base, _BPW)
    copies = [
        pltpu.async_copy(src_hbm.at[sl_in], src_v, sem),
        pltpu.async_copy(cnt_hbm.at[sl_in], cnt_v, sem),
        pltpu.async_copy(lab_hbm.at[sl_in], lab_v, sem),
        pltpu.async_copy(vt_hbm.at[sl_in], vt_v, sem),
    ]
    for cp in copies:
        cp.wait()

    for i in range(_STEPS):
        sl = pl.ds(i * _LANES, _LANES)
        lin_v[sl] = (src_v[sl] * (C * L * T) + cnt_v[sl] * (L * T)
                     + lab_v[sl] * T + vt_v[sl])

    pltpu.async_copy(table_hbm.at[lin_v], out_v, sem).wait()
    pltpu.sync_copy(out_v, out_hbm.at[pl.ds(base, _BPW)])


def kernel(label_balancing_weights_sclt, sources, alt_counts, labels, variant_types):
    table = label_balancing_weights_sclt.reshape(-1)
    return _balancer_gather(table, sources, alt_counts, labels, variant_types)
